# baseline (speedup 1.0000x reference)
import numpy as np
import jax
import jax.numpy as jnp
from jax import lax
from jax.experimental import pallas as pl
from jax.experimental.pallas import tpu as pltpu

N = 8192
H = 64
K = 20
R = 256
C = 512
NCH = N // C
BIG = 1e30
BN_SCALE = 1.0 / np.sqrt(1.0 + 1e-5)

_INTERPRET = False


def _elu(v):
    return jnp.where(v > 0, v, jnp.exp(v) - 1.0)


def _bdot(a, b):
    return jnp.dot(a.astype(jnp.bfloat16), b.astype(jnp.bfloat16),
                   preferred_element_type=jnp.float32)


def _mlp2_kernel(x_ref, w1_ref, w2_ref, o_ref):
    h = _elu(_bdot(x_ref[...], w1_ref[...]))
    o_ref[...] = _elu(_bdot(h, w2_ref[...]))


def _heads_kernel(f_ref, wo1_ref, wo2_ref, wo3_ref, ws1_ref, ws2_ref, ws3_ref,
                  out_ref, split_ref):
    f = f_ref[...]
    o = _elu(_bdot(f, wo1_ref[...]))
    o = _elu(_bdot(o, wo2_ref[...]))
    out_ref[...] = _bdot(o, wo3_ref[...])
    s = _elu(_bdot(f, ws1_ref[...]))
    s = _elu(_bdot(s, ws2_ref[...]))
    split_ref[...] = _bdot(s, ws3_ref[...])


def _knn_agg_kernel(sb_ref, tlo_ref, thi_ref,
                    ft_ref, fr_ref, f3_ref,
                    bcol_ref, brow_ref, wt_ref, gb_ref,
                    o_ref, d_scr, cmin_scr, camin_scr, xj_scr, macc_scr):
    i = pl.program_id(0)
    w0 = sb_ref[tlo_ref[i]]
    w1 = sb_ref[thi_ref[i] + 1]
    c0 = w0 // C
    c1 = (w1 + C - 1) // C

    fr = fr_ref[...]
    frb = fr.astype(jnp.bfloat16)
    sqr = jnp.sum(fr * fr, axis=1, keepdims=True)
    bq = bcol_ref[...]
    wt = wt_ref[...].astype(jnp.bfloat16)
    b = gb_ref[0:1, :]
    gamma = gb_ref[2:3, :]
    beta = gb_ref[4:5, :]

    for j in range(NCH):
        @pl.when(jnp.logical_and(j >= c0, j < c1))
        def _(j=j):
            ftf = ft_ref[:, j * C:(j + 1) * C]
            ftc = ftf.astype(jnp.bfloat16)
            sqc = jnp.sum(ftf * ftf, axis=0, keepdims=True)
            mm = jnp.dot(frb, ftc, preferred_element_type=jnp.float32)
            dch = sqr + sqc - 2.0 * mm
            bc = brow_ref[0:1, j * C:(j + 1) * C]
            d_scr[:, j * C:(j + 1) * C] = jnp.where(bq != bc, BIG, dch)

    macc_scr[...] = jnp.full((R, H), -BIG, jnp.float32)

    def body(_, carry):
        cmin_scr[...] = jnp.full((R, 128), BIG, jnp.float32)
        camin_scr[...] = jnp.full((R, 128), N, jnp.int32)
        for j in range(NCH):
            @pl.when(jnp.logical_and(j >= c0, j < c1))
            def _(j=j):
                dch = d_scr[:, j * C:(j + 1) * C]
                cm = jnp.min(dch, axis=1, keepdims=True)
                colid = lax.broadcasted_iota(jnp.int32, (R, C), 1) + j * C
                ckey = jnp.where(dch == cm, colid, jnp.int32(N))
                cmin_scr[:, j:j + 1] = cm
                camin_scr[:, j:j + 1] = jnp.min(ckey, axis=1, keepdims=True)
        cmins = cmin_scr[:, 0:NCH]
        m = jnp.min(cmins, axis=1, keepdims=True)
        key2 = jnp.where(cmins == m, camin_scr[:, 0:NCH], jnp.int32(N))
        amin = jnp.min(key2, axis=1, keepdims=True)

        nch_w = c1 - c0
        for (w_ch, cap) in ((5, 11), (10, 6)):
            lo = 5 if w_ch == 10 else -1
            @pl.when(jnp.logical_and(nch_w > lo, nch_w <= w_ch))
            def _(w_ch=w_ch, cap=cap):
                base = jnp.minimum(c0, cap) * C
                base = pl.multiple_of(base, C)
                w = w_ch * C
                colid = lax.broadcasted_iota(jnp.int32, (R, w), 1) + base
                onehot = (colid == amin)
                xj_scr[...] = jnp.dot(onehot.astype(jnp.float32),
                                      f3_ref[pl.ds(base, w), :],
                                      preferred_element_type=jnp.float32,
                                      precision=lax.Precision.HIGHEST)
        @pl.when(nch_w > 10)
        def _():
            colid_full = lax.broadcasted_iota(jnp.int32, (R, N), 1)
            onehot_full = (colid_full == amin)
            xj_scr[...] = jnp.dot(onehot_full.astype(jnp.float32), f3_ref[...],
                                  preferred_element_type=jnp.float32,
                                  precision=lax.Precision.HIGHEST)
        xj = xj_scr[...]
        for j in range(NCH):
            @pl.when(jnp.logical_and(j >= c0, j < c1))
            def _(j=j):
                colid = lax.broadcasted_iota(jnp.int32, (R, C), 1) + j * C
                onehot = (colid == amin)
                dch = d_scr[:, j * C:(j + 1) * C]
                d_scr[:, j * C:(j + 1) * C] = jnp.where(onehot, BIG, dch)

        u = jnp.concatenate([fr, xj - fr], axis=1)
        z = jnp.dot(u.astype(jnp.bfloat16), wt,
                    preferred_element_type=jnp.float32) + b
        msg = _elu(z) * (BN_SCALE * gamma) + beta
        valid = m < (BIG * 0.5)
        macc_scr[...] = jnp.where(valid,
                                  jnp.maximum(macc_scr[...], msg),
                                  macc_scr[...])
        return carry

    lax.fori_loop(0, K, body, 0)
    o_ref[...] = macc_scr[...] + fr


def _full(shape):
    return pl.BlockSpec(shape, lambda i: (0, 0))


@jax.jit
def kernel(x, batch, params):
    batch = batch.astype(jnp.int32)
    batch_col = batch.reshape(N, 1)
    batch_row = jnp.broadcast_to(batch.reshape(1, N), (8, N))
    sbounds = jnp.searchsorted(batch, jnp.arange(5, dtype=jnp.int32)
                               ).astype(jnp.int32)
    tile_ids = jnp.arange(N // R, dtype=jnp.int32)
    tlo = batch[tile_ids * R]
    thi = batch[tile_ids * R + (R - 1)]
    sbounds = jnp.pad(sbounds, (0, 3))

    feats = pl.pallas_call(
        _mlp2_kernel,
        grid=(8,),
        in_specs=[pl.BlockSpec((N // 8, 8), lambda i: (i, 0)),
                  _full((8, 32)), _full((32, H))],
        out_specs=pl.BlockSpec((N // 8, H), lambda i: (i, 0)),
        out_shape=jax.ShapeDtypeStruct((N, H), jnp.float32),
        interpret=_INTERPRET,
    )(x, params["W_enc1"].T, params["W_enc2"].T)

    for i in range(4):
        wt = params["conv_W"][i].T
        gb = jnp.concatenate(
            [jnp.broadcast_to(params["conv_b"][i].reshape(1, H), (2, H)),
             jnp.broadcast_to(params["conv_gamma"][i].reshape(1, H), (2, H)),
             jnp.broadcast_to(params["conv_beta"][i].reshape(1, H), (4, H))],
            axis=0)
        ft = feats.T
        feats = pl.pallas_call(
            _knn_agg_kernel,
            grid=(N // R,),
            in_specs=[pl.BlockSpec(memory_space=pltpu.SMEM),
                      pl.BlockSpec(memory_space=pltpu.SMEM),
                      pl.BlockSpec(memory_space=pltpu.SMEM),
                      _full((H, N)),
                      pl.BlockSpec((R, H), lambda i: (i, 0)),
                      _full((N, H)),
                      pl.BlockSpec((R, 1), lambda i: (i, 0)),
                      _full((8, N)),
                      _full((2 * H, H)),
                      _full((8, H))],
            out_specs=pl.BlockSpec((R, H), lambda i: (i, 0)),
            out_shape=jax.ShapeDtypeStruct((N, H), jnp.float32),
            scratch_shapes=[pltpu.VMEM((R, N), jnp.float32),
                            pltpu.VMEM((R, 128), jnp.float32),
                            pltpu.VMEM((R, 128), jnp.int32),
                            pltpu.VMEM((R, H), jnp.float32),
                            pltpu.VMEM((R, H), jnp.float32)],
            interpret=_INTERPRET,
        )(sbounds, tlo, thi, ft, feats, feats,
          batch_col, batch_row, wt, gb)

    out, split_logit = pl.pallas_call(
        _heads_kernel,
        grid=(8,),
        in_specs=[pl.BlockSpec((N // 8, H), lambda i: (i, 0)),
                  _full((H, 64)), _full((64, 32)), _full((32, 8)),
                  _full((H, 64)), _full((64, 32)), _full((32, 1))],
        out_specs=[pl.BlockSpec((N // 8, 8), lambda i: (i, 0)),
                   pl.BlockSpec((N // 8, 1), lambda i: (i, 0))],
        out_shape=[jax.ShapeDtypeStruct((N, 8), jnp.float32),
                   jax.ShapeDtypeStruct((N, 1), jnp.float32)],
        interpret=_INTERPRET,
    )(feats, params["W_o1"].T, params["W_o2"].T, params["W_o3"].T,
      params["W_s1"].T, params["W_s2"].T, params["W_s3"].T)

    return (out, split_logit, batch)


# speedup vs baseline: 2.1120x; 2.1120x over previous
import functools
import numpy as np
import jax
import jax.numpy as jnp
from jax import lax
from jax.experimental import pallas as pl
from jax.experimental.pallas import tpu as pltpu
from jax.experimental.pallas import tpu_sc as plsc

N = 8192
H = 64
K = 20
R = 256
C = 512
NCH = N // C
BIG = 1e30
BN_SCALE = 1.0 / np.sqrt(1.0 + 1e-5)
NW = 32
BPW = N * K // NW
CH = 128

_INTERPRET = False


def _elu(v):
    return jnp.where(v > 0, v, jnp.exp(v) - 1.0)


def _bdot(a, b):
    return jnp.dot(a.astype(jnp.bfloat16), b.astype(jnp.bfloat16),
                   preferred_element_type=jnp.float32)


def _mlp2_kernel(x_ref, w1_ref, w2_ref, o_ref):
    h = _elu(_bdot(x_ref[...], w1_ref[...]))
    o_ref[...] = _elu(_bdot(h, w2_ref[...]))


def _heads_kernel(f_ref, wo1_ref, wo2_ref, wo3_ref, ws1_ref, ws2_ref, ws3_ref,
                  out_ref, split_ref):
    f = f_ref[...]
    o = _elu(_bdot(f, wo1_ref[...]))
    o = _elu(_bdot(o, wo2_ref[...]))
    out_ref[...] = _bdot(o, wo3_ref[...])
    s = _elu(_bdot(f, ws1_ref[...]))
    s = _elu(_bdot(s, ws2_ref[...]))
    split_ref[...] = _bdot(s, ws3_ref[...])


def _knn_idx_kernel(sb_ref, tlo_ref, thi_ref,
                    ft_ref, fr_ref, bcol_ref, brow_ref,
                    idx_ref, vm_ref, d_scr, cmin_scr, camin_scr,
                    idx_scr, vm_scr):
    i = pl.program_id(0)
    w0 = sb_ref[tlo_ref[i]]
    w1 = sb_ref[thi_ref[i] + 1]
    c0 = w0 // C
    c1 = (w1 + C - 1) // C

    fr = fr_ref[...]
    frb = fr.astype(jnp.bfloat16)
    sqr = jnp.sum(fr * fr, axis=1, keepdims=True)
    bq = bcol_ref[...]

    for j in range(NCH):
        @pl.when(jnp.logical_and(j >= c0, j < c1))
        def _(j=j):
            ftf = ft_ref[:, j * C:(j + 1) * C]
            ftc = ftf.astype(jnp.bfloat16)
            sqc = jnp.sum(ftf * ftf, axis=0, keepdims=True)
            mm = jnp.dot(frb, ftc, preferred_element_type=jnp.float32)
            dch = sqr + sqc - 2.0 * mm
            bc = brow_ref[0:1, j * C:(j + 1) * C]
            d_scr[:, j * C:(j + 1) * C] = jnp.where(bq != bc, BIG, dch)

    lane = lax.broadcasted_iota(jnp.int32, (R, 128), 1)

    def body(s, carry):
        cmin_scr[...] = jnp.full((R, 128), BIG, jnp.float32)
        camin_scr[...] = jnp.full((R, 128), N, jnp.int32)
        for j in range(NCH):
            @pl.when(jnp.logical_and(j >= c0, j < c1))
            def _(j=j):
                dch = d_scr[:, j * C:(j + 1) * C]
                cm = jnp.min(dch, axis=1, keepdims=True)
                colid = lax.broadcasted_iota(jnp.int32, (R, C), 1) + j * C
                ckey = jnp.where(dch == cm, colid, jnp.int32(N))
                cmin_scr[:, j:j + 1] = cm
                camin_scr[:, j:j + 1] = jnp.min(ckey, axis=1, keepdims=True)
        cmins = cmin_scr[:, 0:NCH]
        m = jnp.min(cmins, axis=1, keepdims=True)
        key2 = jnp.where(cmins == m, camin_scr[:, 0:NCH], jnp.int32(N))
        amin = jnp.min(key2, axis=1, keepdims=True)

        valid = m < (BIG * 0.5)
        idx_scr[...] = jnp.where(lane == s,
                                 jnp.broadcast_to(jnp.minimum(amin, N - 1),
                                                  (R, 128)),
                                 idx_scr[...])
        vm_scr[...] = jnp.where(lane == s,
                                jnp.broadcast_to(valid, (R, 128)).astype(
                                    jnp.float32),
                                vm_scr[...])
        for j in range(NCH):
            @pl.when(jnp.logical_and(j >= c0, j < c1))
            def _(j=j):
                colid = lax.broadcasted_iota(jnp.int32, (R, C), 1) + j * C
                dch = d_scr[:, j * C:(j + 1) * C]
                d_scr[:, j * C:(j + 1) * C] = jnp.where(colid == amin,
                                                        BIG, dch)
        return carry

    lax.fori_loop(0, K, body, 0)
    idx_ref[...] = idx_scr[:, 0:K]
    vm_ref[...] = vm_scr[:, 0:K]


def _edge_kernel(fr_ref, xj_ref, vm_ref, wt_ref, gb_ref, o_ref):
    fr = fr_ref[...]
    wt = wt_ref[...].astype(jnp.bfloat16)
    b = gb_ref[0:1, :]
    gamma = gb_ref[2:3, :]
    beta = gb_ref[4:5, :]
    macc = jnp.full((R, H), -BIG, jnp.float32)
    vm = vm_ref[...]
    for s in range(K):
        xs = xj_ref[:, s * 128:s * 128 + H]
        u = jnp.concatenate([fr, xs - fr], axis=1)
        z = jnp.dot(u.astype(jnp.bfloat16), wt,
                    preferred_element_type=jnp.float32) + b
        msg = _elu(z) * (BN_SCALE * gamma) + beta
        vs = vm[:, s:s + 1] > 0.5
        macc = jnp.where(vs, jnp.maximum(macc, msg), macc)
    o_ref[...] = macc + fr


def _gather_rows(feats_pad, flat_idx):
    mesh = plsc.VectorSubcoreMesh(core_axis_name="c", subcore_axis_name="s")

    @functools.partial(
        pl.kernel, mesh=mesh,
        out_type=jax.ShapeDtypeStruct((N * K, 128), jnp.float32),
        scratch_types=[pltpu.VMEM((CH,), jnp.int32),
                       pltpu.VMEM((CH, 128), jnp.float32),
                       pltpu.SemaphoreType.DMA],
    )
    def k(feats_hbm, idx_hbm, out_hbm, idx_v, rows_v, sem):
        wid = lax.axis_index("s") * 2 + lax.axis_index("c")
        base = wid * BPW

        def body(t, carry):
            off = base + t * CH
            pltpu.sync_copy(idx_hbm.at[pl.ds(off, CH)], idx_v)
            pltpu.async_copy(feats_hbm.at[idx_v], rows_v, sem).wait()
            pltpu.sync_copy(rows_v, out_hbm.at[pl.ds(off, CH)])
            return carry

        lax.fori_loop(0, BPW // CH, body, 0)

    return k(feats_pad, flat_idx)


def _full(shape):
    return pl.BlockSpec(shape, lambda i: (0, 0))


@jax.jit
def kernel(x, batch, params):
    batch = batch.astype(jnp.int32)
    batch_col = batch.reshape(N, 1)
    batch_row = jnp.broadcast_to(batch.reshape(1, N), (8, N))
    sbounds = jnp.searchsorted(batch, jnp.arange(5, dtype=jnp.int32)
                               ).astype(jnp.int32)
    tile_ids = jnp.arange(N // R, dtype=jnp.int32)
    tlo = batch[tile_ids * R]
    thi = batch[tile_ids * R + (R - 1)]
    sbounds = jnp.pad(sbounds, (0, 3))

    feats = pl.pallas_call(
        _mlp2_kernel,
        grid=(8,),
        in_specs=[pl.BlockSpec((N // 8, 8), lambda i: (i, 0)),
                  _full((8, 32)), _full((32, H))],
        out_specs=pl.BlockSpec((N // 8, H), lambda i: (i, 0)),
        out_shape=jax.ShapeDtypeStruct((N, H), jnp.float32),
        interpret=_INTERPRET,
    )(x, params["W_enc1"].T, params["W_enc2"].T)

    for i in range(4):
        wt = params["conv_W"][i].T
        gb = jnp.concatenate(
            [jnp.broadcast_to(params["conv_b"][i].reshape(1, H), (2, H)),
             jnp.broadcast_to(params["conv_gamma"][i].reshape(1, H), (2, H)),
             jnp.broadcast_to(params["conv_beta"][i].reshape(1, H), (4, H))],
            axis=0)
        ft = feats.T

        idx, vm = pl.pallas_call(
            _knn_idx_kernel,
            grid=(N // R,),
            in_specs=[pl.BlockSpec(memory_space=pltpu.SMEM),
                      pl.BlockSpec(memory_space=pltpu.SMEM),
                      pl.BlockSpec(memory_space=pltpu.SMEM),
                      _full((H, N)),
                      pl.BlockSpec((R, H), lambda i: (i, 0)),
                      pl.BlockSpec((R, 1), lambda i: (i, 0)),
                      _full((8, N))],
            out_specs=[pl.BlockSpec((R, K), lambda i: (i, 0)),
                       pl.BlockSpec((R, K), lambda i: (i, 0))],
            out_shape=[jax.ShapeDtypeStruct((N, K), jnp.int32),
                       jax.ShapeDtypeStruct((N, K), jnp.float32)],
            scratch_shapes=[pltpu.VMEM((R, N), jnp.float32),
                            pltpu.VMEM((R, 128), jnp.float32),
                            pltpu.VMEM((R, 128), jnp.int32),
                            pltpu.VMEM((R, 128), jnp.int32),
                            pltpu.VMEM((R, 128), jnp.float32)],
            interpret=_INTERPRET,
        )(sbounds, tlo, thi, ft, feats, batch_col, batch_row)

        feats_pad = jnp.pad(feats, ((0, 0), (0, 128 - H)))
        xj_flat = _gather_rows(feats_pad, idx.reshape(N * K))
        xj = xj_flat.reshape(N, K * 128)

        feats = pl.pallas_call(
            _edge_kernel,
            grid=(N // R,),
            in_specs=[pl.BlockSpec((R, H), lambda i: (i, 0)),
                      pl.BlockSpec((R, K * 128), lambda i: (i, 0)),
                      pl.BlockSpec((R, K), lambda i: (i, 0)),
                      _full((2 * H, H)),
                      _full((8, H))],
            out_specs=pl.BlockSpec((R, H), lambda i: (i, 0)),
            out_shape=jax.ShapeDtypeStruct((N, H), jnp.float32),
            interpret=_INTERPRET,
        )(feats, xj, vm, wt, gb)

    out, split_logit = pl.pallas_call(
        _heads_kernel,
        grid=(8,),
        in_specs=[pl.BlockSpec((N // 8, H), lambda i: (i, 0)),
                  _full((H, 64)), _full((64, 32)), _full((32, 8)),
                  _full((H, 64)), _full((64, 32)), _full((32, 1))],
        out_specs=[pl.BlockSpec((N // 8, 8), lambda i: (i, 0)),
                   pl.BlockSpec((N // 8, 1), lambda i: (i, 0))],
        out_shape=[jax.ShapeDtypeStruct((N, 8), jnp.float32),
                   jax.ShapeDtypeStruct((N, 1), jnp.float32)],
        interpret=_INTERPRET,
    )(feats, params["W_o1"].T, params["W_o2"].T, params["W_o3"].T,
      params["W_s1"].T, params["W_s2"].T, params["W_s3"].T)

    return (out, split_logit, batch)


# speedup vs baseline: 2.2980x; 1.0881x over previous
import functools
import numpy as np
import jax
import jax.numpy as jnp
from jax import lax
from jax.experimental import pallas as pl
from jax.experimental.pallas import tpu as pltpu
from jax.experimental.pallas import tpu_sc as plsc

N = 8192
H = 64
K = 20
R = 256
C = 512
NCH = N // C
BIG = 1e30
BN_SCALE = 1.0 / np.sqrt(1.0 + 1e-5)
NW = 32
BPW = N * K // NW
CH = 128

_INTERPRET = False


def _elu(v):
    return jnp.where(v > 0, v, jnp.exp(v) - 1.0)


def _bdot(a, b):
    return jnp.dot(a.astype(jnp.bfloat16), b.astype(jnp.bfloat16),
                   preferred_element_type=jnp.float32)


def _mlp2_kernel(x_ref, w1_ref, w2_ref, o_ref):
    h = _elu(_bdot(x_ref[...], w1_ref[...]))
    o_ref[...] = _elu(_bdot(h, w2_ref[...]))


def _heads_kernel(f_ref, wo1_ref, wo2_ref, wo3_ref, ws1_ref, ws2_ref, ws3_ref,
                  out_ref, split_ref):
    f = f_ref[...]
    o = _elu(_bdot(f, wo1_ref[...]))
    o = _elu(_bdot(o, wo2_ref[...]))
    out_ref[...] = _bdot(o, wo3_ref[...])
    s = _elu(_bdot(f, ws1_ref[...]))
    s = _elu(_bdot(s, ws2_ref[...]))
    split_ref[...] = _bdot(s, ws3_ref[...])


def _knn_idx_kernel(sb_ref, tlo_ref, thi_ref,
                    ft_ref, fr_ref, bcol_ref, brow_ref,
                    idx_ref, vm_ref, d_scr, cmin_scr, camin_scr,
                    idx_scr, vm_scr):
    i = pl.program_id(0)
    w0 = sb_ref[tlo_ref[i]]
    w1 = sb_ref[thi_ref[i] + 1]
    c0 = w0 // C
    c1 = (w1 + C - 1) // C

    fr = fr_ref[...]
    frb = fr.astype(jnp.bfloat16)
    sqr = jnp.sum(fr * fr, axis=1, keepdims=True)
    bq = bcol_ref[...]

    for j in range(NCH):
        @pl.when(jnp.logical_and(j >= c0, j < c1))
        def _(j=j):
            ftf = ft_ref[:, j * C:(j + 1) * C]
            ftc = ftf.astype(jnp.bfloat16)
            sqc = jnp.sum(ftf * ftf, axis=0, keepdims=True)
            mm = jnp.dot(frb, ftc, preferred_element_type=jnp.float32)
            dch = sqr + sqc - 2.0 * mm
            bc = brow_ref[0:1, j * C:(j + 1) * C]
            d_scr[:, j * C:(j + 1) * C] = jnp.where(bq != bc, BIG, dch)

    lane = lax.broadcasted_iota(jnp.int32, (R, 128), 1)

    def body(s, aprev):
        cmin_scr[...] = jnp.full((R, 128), BIG, jnp.float32)
        camin_scr[...] = jnp.full((R, 128), N, jnp.int32)
        for j in range(NCH):
            @pl.when(jnp.logical_and(j >= c0, j < c1))
            def _(j=j):
                colid = lax.broadcasted_iota(jnp.int32, (R, C), 1) + j * C
                dch = jnp.where(colid == aprev, BIG,
                                d_scr[:, j * C:(j + 1) * C])
                d_scr[:, j * C:(j + 1) * C] = dch
                cm = jnp.min(dch, axis=1, keepdims=True)
                ckey = jnp.where(dch == cm, colid, jnp.int32(N))
                cmin_scr[:, j:j + 1] = cm
                camin_scr[:, j:j + 1] = jnp.min(ckey, axis=1, keepdims=True)
        cmins = cmin_scr[:, 0:NCH]
        m = jnp.min(cmins, axis=1, keepdims=True)
        key2 = jnp.where(cmins == m, camin_scr[:, 0:NCH], jnp.int32(N))
        amin = jnp.min(key2, axis=1, keepdims=True)

        valid = m < (BIG * 0.5)
        idx_scr[...] = jnp.where(lane == s,
                                 jnp.broadcast_to(jnp.minimum(amin, N - 1),
                                                  (R, 128)),
                                 idx_scr[...])
        vm_scr[...] = jnp.where(lane == s,
                                jnp.broadcast_to(valid, (R, 128)).astype(
                                    jnp.float32),
                                vm_scr[...])
        return amin

    lax.fori_loop(0, K, body, jnp.full((R, 1), -1, jnp.int32))
    idx_ref[...] = idx_scr[:, 0:K]
    vm_ref[...] = vm_scr[:, 0:K]


def _edge_kernel(fr_ref, xj_ref, vm_ref, wt_ref, gb_ref, o_ref):
    fr = fr_ref[...]
    wt = wt_ref[...].astype(jnp.bfloat16)
    b = gb_ref[0:1, :]
    gamma = gb_ref[2:3, :]
    beta = gb_ref[4:5, :]
    macc = jnp.full((R, H), -BIG, jnp.float32)
    vm = vm_ref[...]
    for s in range(K):
        xs = xj_ref[:, s * 128:s * 128 + H]
        u = jnp.concatenate([fr, xs - fr], axis=1)
        z = jnp.dot(u.astype(jnp.bfloat16), wt,
                    preferred_element_type=jnp.float32) + b
        msg = _elu(z) * (BN_SCALE * gamma) + beta
        vs = vm[:, s:s + 1] > 0.5
        macc = jnp.where(vs, jnp.maximum(macc, msg), macc)
    o_ref[...] = macc + fr


def _gather_rows(feats_pad, flat_idx):
    mesh = plsc.VectorSubcoreMesh(core_axis_name="c", subcore_axis_name="s")

    @functools.partial(
        pl.kernel, mesh=mesh,
        out_type=jax.ShapeDtypeStruct((N * K, 128), jnp.float32),
        scratch_types=[pltpu.VMEM((CH,), jnp.int32),
                       pltpu.VMEM((CH, 128), jnp.float32),
                       pltpu.SemaphoreType.DMA],
    )
    def k(feats_hbm, idx_hbm, out_hbm, idx_v, rows_v, sem):
        wid = lax.axis_index("s") * 2 + lax.axis_index("c")
        base = wid * BPW

        def body(t, carry):
            off = base + t * CH
            pltpu.sync_copy(idx_hbm.at[pl.ds(off, CH)], idx_v)
            pltpu.async_copy(feats_hbm.at[idx_v], rows_v, sem).wait()
            pltpu.sync_copy(rows_v, out_hbm.at[pl.ds(off, CH)])
            return carry

        lax.fori_loop(0, BPW // CH, body, 0)

    return k(feats_pad, flat_idx)


def _full(shape):
    return pl.BlockSpec(shape, lambda i: (0, 0))


@jax.jit
def kernel(x, batch, params):
    batch = batch.astype(jnp.int32)
    batch_col = batch.reshape(N, 1)
    batch_row = jnp.broadcast_to(batch.reshape(1, N), (8, N))
    sbounds = jnp.searchsorted(batch, jnp.arange(5, dtype=jnp.int32)
                               ).astype(jnp.int32)
    tile_ids = jnp.arange(N // R, dtype=jnp.int32)
    tlo = batch[tile_ids * R]
    thi = batch[tile_ids * R + (R - 1)]
    sbounds = jnp.pad(sbounds, (0, 3))

    feats = pl.pallas_call(
        _mlp2_kernel,
        grid=(8,),
        in_specs=[pl.BlockSpec((N // 8, 8), lambda i: (i, 0)),
                  _full((8, 32)), _full((32, H))],
        out_specs=pl.BlockSpec((N // 8, H), lambda i: (i, 0)),
        out_shape=jax.ShapeDtypeStruct((N, H), jnp.float32),
        interpret=_INTERPRET,
    )(x, params["W_enc1"].T, params["W_enc2"].T)

    for i in range(4):
        wt = params["conv_W"][i].T
        gb = jnp.concatenate(
            [jnp.broadcast_to(params["conv_b"][i].reshape(1, H), (2, H)),
             jnp.broadcast_to(params["conv_gamma"][i].reshape(1, H), (2, H)),
             jnp.broadcast_to(params["conv_beta"][i].reshape(1, H), (4, H))],
            axis=0)
        ft = feats.T

        idx, vm = pl.pallas_call(
            _knn_idx_kernel,
            grid=(N // R,),
            in_specs=[pl.BlockSpec(memory_space=pltpu.SMEM),
                      pl.BlockSpec(memory_space=pltpu.SMEM),
                      pl.BlockSpec(memory_space=pltpu.SMEM),
                      _full((H, N)),
                      pl.BlockSpec((R, H), lambda i: (i, 0)),
                      pl.BlockSpec((R, 1), lambda i: (i, 0)),
                      _full((8, N))],
            out_specs=[pl.BlockSpec((R, K), lambda i: (i, 0)),
                       pl.BlockSpec((R, K), lambda i: (i, 0))],
            out_shape=[jax.ShapeDtypeStruct((N, K), jnp.int32),
                       jax.ShapeDtypeStruct((N, K), jnp.float32)],
            scratch_shapes=[pltpu.VMEM((R, N), jnp.float32),
                            pltpu.VMEM((R, 128), jnp.float32),
                            pltpu.VMEM((R, 128), jnp.int32),
                            pltpu.VMEM((R, 128), jnp.int32),
                            pltpu.VMEM((R, 128), jnp.float32)],
            interpret=_INTERPRET,
        )(sbounds, tlo, thi, ft, feats, batch_col, batch_row)

        feats_pad = jnp.pad(feats, ((0, 0), (0, 128 - H)))
        xj_flat = _gather_rows(feats_pad, idx.reshape(N * K))
        xj = xj_flat.reshape(N, K * 128)

        feats = pl.pallas_call(
            _edge_kernel,
            grid=(N // R,),
            in_specs=[pl.BlockSpec((R, H), lambda i: (i, 0)),
                      pl.BlockSpec((R, K * 128), lambda i: (i, 0)),
                      pl.BlockSpec((R, K), lambda i: (i, 0)),
                      _full((2 * H, H)),
                      _full((8, H))],
            out_specs=pl.BlockSpec((R, H), lambda i: (i, 0)),
            out_shape=jax.ShapeDtypeStruct((N, H), jnp.float32),
            interpret=_INTERPRET,
        )(feats, xj, vm, wt, gb)

    out, split_logit = pl.pallas_call(
        _heads_kernel,
        grid=(8,),
        in_specs=[pl.BlockSpec((N // 8, H), lambda i: (i, 0)),
                  _full((H, 64)), _full((64, 32)), _full((32, 8)),
                  _full((H, 64)), _full((64, 32)), _full((32, 1))],
        out_specs=[pl.BlockSpec((N // 8, 8), lambda i: (i, 0)),
                   pl.BlockSpec((N // 8, 1), lambda i: (i, 0))],
        out_shape=[jax.ShapeDtypeStruct((N, 8), jnp.float32),
                   jax.ShapeDtypeStruct((N, 1), jnp.float32)],
        interpret=_INTERPRET,
    )(feats, params["W_o1"].T, params["W_o2"].T, params["W_o3"].T,
      params["W_s1"].T, params["W_s2"].T, params["W_s3"].T)

    return (out, split_logit, batch)


# speedup vs baseline: 3.1808x; 1.3842x over previous
import functools
import numpy as np
import jax
import jax.numpy as jnp
from jax import lax
from jax.experimental import pallas as pl
from jax.experimental.pallas import tpu as pltpu
from jax.experimental.pallas import tpu_sc as plsc

N = 8192
H = 64
K = 20
R = 256
C = 512
NCH = N // C
BIG = 1e30
BN_SCALE = 1.0 / np.sqrt(1.0 + 1e-5)
NW = 32
BPW = N * K // NW
CH = 128

_INTERPRET = False


def _elu(v):
    return jnp.where(v > 0, v, jnp.exp(v) - 1.0)


def _bdot(a, b):
    return jnp.dot(a.astype(jnp.bfloat16), b.astype(jnp.bfloat16),
                   preferred_element_type=jnp.float32)


def _mlp2_kernel(x_ref, w1_ref, w2_ref, o_ref):
    h = _elu(_bdot(x_ref[...], w1_ref[...]))
    o_ref[...] = _elu(_bdot(h, w2_ref[...]))


def _heads_kernel(f_ref, wo1_ref, wo2_ref, wo3_ref, ws1_ref, ws2_ref, ws3_ref,
                  out_ref, split_ref):
    f = f_ref[...]
    o = _elu(_bdot(f, wo1_ref[...]))
    o = _elu(_bdot(o, wo2_ref[...]))
    out_ref[...] = _bdot(o, wo3_ref[...])
    s = _elu(_bdot(f, ws1_ref[...]))
    s = _elu(_bdot(s, ws2_ref[...]))
    split_ref[...] = _bdot(s, ws3_ref[...])


def _knn_idx_kernel(sb_ref, tlo_ref, thi_ref,
                    ft_ref, fr_ref, bcol_ref, brow_ref,
                    idx_ref, vm_ref, d_scr, cmin_scr, camin_scr,
                    idx_scr, vm_scr):
    i = pl.program_id(0)
    w0 = sb_ref[tlo_ref[i]]
    w1 = sb_ref[thi_ref[i] + 1]
    c0 = w0 // C
    c1 = (w1 + C - 1) // C

    fr = fr_ref[...]
    frb = fr.astype(jnp.bfloat16)
    sqr = jnp.sum(fr * fr, axis=1, keepdims=True)
    bq = bcol_ref[...]

    nch_w = c1 - c0
    tb = jnp.where(nch_w <= 5, jnp.minimum(c0, NCH - 5),
                   jnp.where(nch_w <= 10, jnp.minimum(c0, NCH - 10), 0))
    tw = jnp.where(nch_w <= 5, 5, jnp.where(nch_w <= 10, 10, NCH))
    for j in range(NCH):
        in_win = jnp.logical_and(j >= c0, j < c1)
        in_tier = jnp.logical_and(j >= tb, j < tb + tw)
        @pl.when(in_win)
        def _(j=j):
            ftf = ft_ref[:, j * C:(j + 1) * C]
            ftc = ftf.astype(jnp.bfloat16)
            sqc = jnp.sum(ftf * ftf, axis=0, keepdims=True)
            mm = jnp.dot(frb, ftc, preferred_element_type=jnp.float32)
            dch = sqr + sqc - 2.0 * mm
            bc = brow_ref[0:1, j * C:(j + 1) * C]
            d_scr[:, j * C:(j + 1) * C] = jnp.where(bq != bc, BIG, dch)
        @pl.when(jnp.logical_and(in_tier, jnp.logical_not(in_win)))
        def _(j=j):
            d_scr[:, j * C:(j + 1) * C] = jnp.full((R, C), BIG, jnp.float32)

    lane = lax.broadcasted_iota(jnp.int32, (R, 128), 1)

    def body(s, aprev):
        for (w_ch, lo) in ((5, -1), (10, 5), (NCH, 10)):
            @pl.when(jnp.logical_and(nch_w > lo, nch_w <= w_ch))
            def _(w_ch=w_ch):
                base = tb * C
                base = pl.multiple_of(base, C)
                w = w_ch * C
                colid = lax.broadcasted_iota(jnp.int32, (R, w), 1) + base
                dw = jnp.where(colid == aprev, BIG, d_scr[:, pl.ds(base, w)])
                d_scr[:, pl.ds(base, w)] = dw
                cm = jnp.min(dw, axis=1, keepdims=True)
                ckey = jnp.where(dw == cm, colid, jnp.int32(N))
                cmin_scr[:, 0:1] = cm
                camin_scr[:, 0:1] = jnp.min(ckey, axis=1, keepdims=True)
        m = cmin_scr[:, 0:1]
        amin = camin_scr[:, 0:1]

        valid = m < (BIG * 0.5)
        idx_scr[...] = jnp.where(lane == s,
                                 jnp.broadcast_to(jnp.minimum(amin, N - 1),
                                                  (R, 128)),
                                 idx_scr[...])
        vm_scr[...] = jnp.where(lane == s,
                                jnp.broadcast_to(valid, (R, 128)).astype(
                                    jnp.float32),
                                vm_scr[...])
        return amin

    lax.fori_loop(0, K, body, jnp.full((R, 1), -1, jnp.int32))
    idx_ref[...] = idx_scr[:, 0:K]
    vm_ref[...] = vm_scr[:, 0:K]


def _edge_kernel(fr_ref, xj_ref, vm_ref, wt_ref, gb_ref, o_ref):
    fr = fr_ref[...]
    wt = wt_ref[...].astype(jnp.bfloat16)
    b = gb_ref[0:1, :]
    gamma = gb_ref[2:3, :]
    beta = gb_ref[4:5, :]
    macc = jnp.full((R, H), -BIG, jnp.float32)
    vm = vm_ref[...]
    for s in range(K):
        xs = xj_ref[:, s * 128:s * 128 + H]
        u = jnp.concatenate([fr, xs - fr], axis=1)
        z = jnp.dot(u.astype(jnp.bfloat16), wt,
                    preferred_element_type=jnp.float32) + b
        msg = _elu(z) * (BN_SCALE * gamma) + beta
        vs = vm[:, s:s + 1] > 0.5
        macc = jnp.where(vs, jnp.maximum(macc, msg), macc)
    o_ref[...] = macc + fr


def _gather_rows(feats_pad, flat_idx):
    mesh = plsc.VectorSubcoreMesh(core_axis_name="c", subcore_axis_name="s")

    @functools.partial(
        pl.kernel, mesh=mesh,
        out_type=jax.ShapeDtypeStruct((N * K, 128), jnp.float32),
        scratch_types=[pltpu.VMEM((CH,), jnp.int32),
                       pltpu.VMEM((CH, 128), jnp.float32),
                       pltpu.SemaphoreType.DMA],
    )
    def k(feats_hbm, idx_hbm, out_hbm, idx_v, rows_v, sem):
        wid = lax.axis_index("s") * 2 + lax.axis_index("c")
        base = wid * BPW

        def body(t, carry):
            off = base + t * CH
            pltpu.sync_copy(idx_hbm.at[pl.ds(off, CH)], idx_v)
            pltpu.async_copy(feats_hbm.at[idx_v], rows_v, sem).wait()
            pltpu.sync_copy(rows_v, out_hbm.at[pl.ds(off, CH)])
            return carry

        lax.fori_loop(0, BPW // CH, body, 0)

    return k(feats_pad, flat_idx)


def _full(shape):
    return pl.BlockSpec(shape, lambda i: (0, 0))


@jax.jit
def kernel(x, batch, params):
    batch = batch.astype(jnp.int32)
    batch_col = batch.reshape(N, 1)
    batch_row = jnp.broadcast_to(batch.reshape(1, N), (8, N))
    sbounds = jnp.searchsorted(batch, jnp.arange(5, dtype=jnp.int32)
                               ).astype(jnp.int32)
    tile_ids = jnp.arange(N // R, dtype=jnp.int32)
    tlo = batch[tile_ids * R]
    thi = batch[tile_ids * R + (R - 1)]
    sbounds = jnp.pad(sbounds, (0, 3))

    feats = pl.pallas_call(
        _mlp2_kernel,
        grid=(8,),
        in_specs=[pl.BlockSpec((N // 8, 8), lambda i: (i, 0)),
                  _full((8, 32)), _full((32, H))],
        out_specs=pl.BlockSpec((N // 8, H), lambda i: (i, 0)),
        out_shape=jax.ShapeDtypeStruct((N, H), jnp.float32),
        interpret=_INTERPRET,
    )(x, params["W_enc1"].T, params["W_enc2"].T)

    for i in range(4):
        wt = params["conv_W"][i].T
        gb = jnp.concatenate(
            [jnp.broadcast_to(params["conv_b"][i].reshape(1, H), (2, H)),
             jnp.broadcast_to(params["conv_gamma"][i].reshape(1, H), (2, H)),
             jnp.broadcast_to(params["conv_beta"][i].reshape(1, H), (4, H))],
            axis=0)
        ft = feats.T

        idx, vm = pl.pallas_call(
            _knn_idx_kernel,
            grid=(N // R,),
            in_specs=[pl.BlockSpec(memory_space=pltpu.SMEM),
                      pl.BlockSpec(memory_space=pltpu.SMEM),
                      pl.BlockSpec(memory_space=pltpu.SMEM),
                      _full((H, N)),
                      pl.BlockSpec((R, H), lambda i: (i, 0)),
                      pl.BlockSpec((R, 1), lambda i: (i, 0)),
                      _full((8, N))],
            out_specs=[pl.BlockSpec((R, K), lambda i: (i, 0)),
                       pl.BlockSpec((R, K), lambda i: (i, 0))],
            out_shape=[jax.ShapeDtypeStruct((N, K), jnp.int32),
                       jax.ShapeDtypeStruct((N, K), jnp.float32)],
            scratch_shapes=[pltpu.VMEM((R, N), jnp.float32),
                            pltpu.VMEM((R, 128), jnp.float32),
                            pltpu.VMEM((R, 128), jnp.int32),
                            pltpu.VMEM((R, 128), jnp.int32),
                            pltpu.VMEM((R, 128), jnp.float32)],
            interpret=_INTERPRET,
        )(sbounds, tlo, thi, ft, feats, batch_col, batch_row)

        feats_pad = jnp.pad(feats, ((0, 0), (0, 128 - H)))
        xj_flat = _gather_rows(feats_pad, idx.reshape(N * K))
        xj = xj_flat.reshape(N, K * 128)

        feats = pl.pallas_call(
            _edge_kernel,
            grid=(N // R,),
            in_specs=[pl.BlockSpec((R, H), lambda i: (i, 0)),
                      pl.BlockSpec((R, K * 128), lambda i: (i, 0)),
                      pl.BlockSpec((R, K), lambda i: (i, 0)),
                      _full((2 * H, H)),
                      _full((8, H))],
            out_specs=pl.BlockSpec((R, H), lambda i: (i, 0)),
            out_shape=jax.ShapeDtypeStruct((N, H), jnp.float32),
            interpret=_INTERPRET,
        )(feats, xj, vm, wt, gb)

    out, split_logit = pl.pallas_call(
        _heads_kernel,
        grid=(8,),
        in_specs=[pl.BlockSpec((N // 8, H), lambda i: (i, 0)),
                  _full((H, 64)), _full((64, 32)), _full((32, 8)),
                  _full((H, 64)), _full((64, 32)), _full((32, 1))],
        out_specs=[pl.BlockSpec((N // 8, 8), lambda i: (i, 0)),
                   pl.BlockSpec((N // 8, 1), lambda i: (i, 0))],
        out_shape=[jax.ShapeDtypeStruct((N, 8), jnp.float32),
                   jax.ShapeDtypeStruct((N, 1), jnp.float32)],
        interpret=_INTERPRET,
    )(feats, params["W_o1"].T, params["W_o2"].T, params["W_o3"].T,
      params["W_s1"].T, params["W_s2"].T, params["W_s3"].T)

    return (out, split_logit, batch)
